# baseline (speedup 1.0000x reference)
import functools

import jax
import jax.numpy as jnp
from jax.experimental import pallas as pl
from jax.experimental.pallas import tpu as pltpu
from jax.experimental.pallas import tpu_sc as plsc

SKIP = 0.125
_INT_MIN = -2147483648


def _router_copy_kernel(x_ref, wr_ref, out_ref, log_ref):
    xb = x_ref[0]
    out_ref[0] = xb
    log_ref[0] = jnp.sum(xb * wr_ref[...], axis=1, keepdims=True)


def _copy_and_logits(x, Wr, bs):
    B, S, D = x.shape
    res, logits3 = pl.pallas_call(
        _router_copy_kernel,
        grid=(B, S // bs),
        in_specs=[
            pl.BlockSpec((1, bs, D), lambda b, sb: (b, sb, 0)),
            pl.BlockSpec((1, D), lambda b, sb: (0, 0)),
        ],
        out_specs=[
            pl.BlockSpec((1, bs, D), lambda b, sb: (b, sb, 0)),
            pl.BlockSpec((1, bs, 1), lambda b, sb: (b, sb, 0)),
        ],
        out_shape=[
            jax.ShapeDtypeStruct((B, S, D), jnp.float32),
            jax.ShapeDtypeStruct((B, S, 1), jnp.float32),
        ],
    )(x, Wr.reshape(1, D))
    return res, logits3


def _cumsum_excl(a, u_cols, l_rows):
    colcum = jnp.dot(a, u_cols, preferred_element_type=jnp.float32)
    rowtot = colcum[:, -1:]
    off = jnp.dot(l_rows, rowtot, preferred_element_type=jnp.float32)
    return colcum + off - a


def _route_kernel(log_ref, tok_ref, rw_ref, gtok_ref, *, k, S):
    L3 = log_ref[...]
    B, R, C = L3.shape

    int_min = jnp.int32(_INT_MIN)
    u = jax.lax.bitcast_convert_type(L3, jnp.int32)
    o = jnp.where(u < 0, u ^ jnp.int32(0x7FFFFFFF), u)
    bb = o ^ int_min

    def radix_body(t, carry):
        prefix, gcnt = carry
        bitpos = 31 - t
        bit = jnp.int32(1) << bitpos
        dm = -(bit << 1)
        cand = prefix | bit
        is_cand = (bb & (dm | bit)) == cand
        c1 = jnp.sum(is_cand.astype(jnp.int32), axis=(1, 2), keepdims=True)
        take = gcnt + c1 >= k
        prefix = jnp.where(take, cand, prefix)
        gcnt = jnp.where(take, gcnt, gcnt + c1)
        return prefix, gcnt

    z11 = jnp.zeros((B, 1, 1), jnp.int32)
    prefix, _ = jax.lax.fori_loop(0, 32, radix_body, (z11, z11))
    t_o = prefix ^ int_min

    gt = o > t_o
    eq = o == t_o
    G = jnp.sum(gt.astype(jnp.int32), axis=(1, 2), keepdims=True)
    needf = (k - G).astype(jnp.float32)
    m = jnp.max(L3, axis=(1, 2), keepdims=True)

    iota_c = jax.lax.broadcasted_iota(jnp.int32, (C, C), 1)
    iota_r = jax.lax.broadcasted_iota(jnp.int32, (C, C), 0)
    u_cols = (iota_r <= iota_c).astype(jnp.float32)
    ri = jax.lax.broadcasted_iota(jnp.int32, (R, R), 0)
    ci = jax.lax.broadcasted_iota(jnp.int32, (R, R), 1)
    l_rows = (ci < ri).astype(jnp.float32)
    idx = (jax.lax.broadcasted_iota(jnp.int32, (R, C), 0) * C
           + jax.lax.broadcasted_iota(jnp.int32, (R, C), 1)).astype(jnp.float32)
    iota_j = jax.lax.broadcasted_iota(jnp.int32, (k, C), 0).astype(jnp.float32)

    for b in range(B):
        rank_eq = _cumsum_excl(eq[b].astype(jnp.float32), u_cols, l_rows)
        sel = gt[b] | (eq[b] & (rank_eq < needf[b, 0, 0]))
        self32 = sel.astype(jnp.float32)
        p = _cumsum_excl(self32, u_cols, l_rows)
        e = jnp.exp(L3[b] - m[b, 0, 0]) * self32
        selidx = self32 * idx
        acc = jnp.zeros((2, k), jnp.float32)
        for r in range(R):
            oh = (iota_j == p[r:r + 1]).astype(jnp.float32)
            a2 = jnp.concatenate([selidx[r:r + 1], e[r:r + 1]], 0)
            acc = acc + jax.lax.dot_general(
                a2, oh, (((1,), (1,)), ((), ())),
                preferred_element_type=jnp.float32)
        z = jnp.sum(e)
        toks = acc[0:1].astype(jnp.int32)
        tok_ref[b] = toks
        rw_ref[b] = acc[1:2] / z
        gtok_ref[b] = toks + b * S


def _route(logits3, k):
    B, S, _ = logits3.shape
    R = 8
    C = S // R
    logr = logits3.reshape(B, R, C)
    tokens3, rw3, gtok3 = pl.pallas_call(
        functools.partial(_route_kernel, k=k, S=S),
        out_shape=[
            jax.ShapeDtypeStruct((B, 1, k), jnp.int32),
            jax.ShapeDtypeStruct((B, 1, k), jnp.float32),
            jax.ShapeDtypeStruct((B, 1, k), jnp.int32),
        ],
    )(logr)
    return tokens3.reshape(B, k), rw3.reshape(B, k, 1), gtok3.reshape(B * k)


def _sc_gather(xf, gtok, D):
    total = gtok.shape[0]
    NC, NS = 2, 16
    NW = NC * NS
    rows_w = total // NW
    CH = min(32, rows_w)
    nch = rows_w // CH
    mesh = plsc.VectorSubcoreMesh(core_axis_name="c", subcore_axis_name="s",
                                  num_cores=NC, num_subcores=NS)

    @functools.partial(
        pl.kernel,
        out_type=jax.ShapeDtypeStruct((total, D), jnp.float32),
        mesh=mesh,
        scratch_types=[
            pltpu.VMEM((CH,), jnp.int32),
            pltpu.VMEM((CH, D), jnp.float32),
            pltpu.SemaphoreType.DMA,
        ],
    )
    def gk(x_hbm, tok_hbm, out_hbm, idx_v, rows_v, sem):
        wid = jax.lax.axis_index("s") * NC + jax.lax.axis_index("c")
        base = wid * rows_w
        for c in range(nch):
            g0 = base + c * CH
            pltpu.sync_copy(tok_hbm.at[pl.ds(g0, CH)], idx_v)
            pltpu.async_copy(x_hbm.at[idx_v], rows_v, sem).wait()
            pltpu.sync_copy(rows_v, out_hbm.at[pl.ds(g0, CH), :])

    return gk(xf, gtok)


def _moe_kernel(tok_ref, filt_ref, wl_ref, bl_ref, rw_ref, res_any, out_any,
                ys, sems, *, bm, nsteps, bofs):
    b = pl.program_id(0)
    jb = pl.program_id(1)
    nj = pl.num_programs(1)
    s = b * nj + jb
    base = jb * bm
    buf = jax.lax.rem(s, 2)

    def _drain(which):
        def s_wait(j, _):
            pltpu.make_async_copy(ys.at[which, pl.ds(0, 1), :],
                                  out_any.at[0, pl.ds(0, 1), :],
                                  sems.at[which]).wait()
            return 0
        jax.lax.fori_loop(0, bm, s_wait, 0, unroll=16)

    @pl.when(s >= 2)
    def _():
        _drain(buf)

    xb = filt_ref[0]
    acc = jnp.dot(xb.astype(jnp.bfloat16),
                  wl_ref[...].astype(jnp.bfloat16),
                  preferred_element_type=jnp.float32)
    ys[buf] = xb + rw_ref[0] * (acc + bl_ref[...])

    def s_start(j, _):
        t = tok_ref[b, base + j]
        pltpu.make_async_copy(ys.at[buf, pl.ds(j, 1), :],
                              out_any.at[bofs + b, pl.ds(t, 1), :],
                              sems.at[buf]).start()
        return 0

    jax.lax.fori_loop(0, bm, s_start, 0, unroll=16)

    @pl.when(s == nsteps - 1)
    def _():
        _drain(buf)
        if nsteps >= 2:
            _drain(1 - buf)


def _moe(tokens, filt, Wl, bl2, rwk, res0, bm, bofs):
    Bg, S, D = res0.shape
    B, k = tokens.shape
    nsteps = B * (k // bm)
    grid_spec = pltpu.PrefetchScalarGridSpec(
        num_scalar_prefetch=1,
        grid=(B, k // bm),
        in_specs=[
            pl.BlockSpec((1, bm, D), lambda b, j, tok: (b, j, 0)),
            pl.BlockSpec((D, D), lambda b, j, tok: (0, 0)),
            pl.BlockSpec((1, D), lambda b, j, tok: (0, 0)),
            pl.BlockSpec((1, bm, 1), lambda b, j, tok: (b, j, 0)),
            pl.BlockSpec(memory_space=pl.MemorySpace.ANY),
        ],
        out_specs=pl.BlockSpec(memory_space=pl.MemorySpace.ANY),
        scratch_shapes=[
            pltpu.VMEM((2, bm, D), jnp.float32),
            pltpu.SemaphoreType.DMA((2,)),
        ],
    )
    return pl.pallas_call(
        functools.partial(_moe_kernel, bm=bm, nsteps=nsteps, bofs=bofs),
        grid_spec=grid_spec,
        out_shape=jax.ShapeDtypeStruct((Bg, S, D), jnp.float32),
        input_output_aliases={5: 0},
        compiler_params=pltpu.CompilerParams(
            dimension_semantics=("arbitrary", "arbitrary"),
        ),
    )(tokens, filt, Wl, bl2, rwk, res0)


def kernel(x, Wr, br, Wl, bl):
    B, S, D = x.shape
    k = int(S * SKIP) or 1
    res0, logits3 = _copy_and_logits(x, Wr, bs=1024)
    tokens, rwk, gtok = _route(logits3, k)
    xf = x.reshape(B * S, D)
    bl2 = bl.reshape(1, D)
    res = res0
    for b in range(B):
        filt_b = _sc_gather(xf, gtok[b * k:(b + 1) * k], D)
        res = _moe(tokens[b:b + 1], filt_b.reshape(1, k, D), Wl, bl2,
                   rwk[b:b + 1], res, bm=512, bofs=b)
    return res


# speedup vs baseline: 1.1268x; 1.1268x over previous
import functools

import jax
import jax.numpy as jnp
from jax.experimental import pallas as pl
from jax.experimental.pallas import tpu as pltpu
from jax.experimental.pallas import tpu_sc as plsc

SKIP = 0.125
_INT_MIN = -2147483648


def _router_copy_kernel(x_ref, wr_ref, out_ref, log_ref):
    xb = x_ref[0]
    out_ref[0] = xb
    log_ref[0] = jnp.sum(xb * wr_ref[...], axis=1, keepdims=True)


def _copy_and_logits(x, Wr, bs):
    B, S, D = x.shape
    res, logits3 = pl.pallas_call(
        _router_copy_kernel,
        grid=(B, S // bs),
        in_specs=[
            pl.BlockSpec((1, bs, D), lambda b, sb: (b, sb, 0)),
            pl.BlockSpec((1, D), lambda b, sb: (0, 0)),
        ],
        out_specs=[
            pl.BlockSpec((1, bs, D), lambda b, sb: (b, sb, 0)),
            pl.BlockSpec((1, bs, 1), lambda b, sb: (b, sb, 0)),
        ],
        out_shape=[
            jax.ShapeDtypeStruct((B, S, D), jnp.float32),
            jax.ShapeDtypeStruct((B, S, 1), jnp.float32),
        ],
    )(x, Wr.reshape(1, D))
    return res, logits3


def _cumsum_excl(a, u_cols, l_rows):
    colcum = jnp.dot(a, u_cols, preferred_element_type=jnp.float32)
    rowtot = colcum[:, -1:]
    off = jnp.dot(l_rows, rowtot, preferred_element_type=jnp.float32)
    return colcum + off - a


def _route_kernel(log_ref, tok_ref, rw_ref, gtok_ref, *, k, S):
    L3 = log_ref[...]
    B, R, C = L3.shape

    int_min = jnp.int32(_INT_MIN)
    u = jax.lax.bitcast_convert_type(L3, jnp.int32)
    o = jnp.where(u < 0, u ^ jnp.int32(0x7FFFFFFF), u)
    bb = o ^ int_min

    def radix_body(t, carry):
        prefix, gcnt = carry
        bitpos = 31 - t
        bit = jnp.int32(1) << bitpos
        dm = -(bit << 1)
        cand = prefix | bit
        is_cand = (bb & (dm | bit)) == cand
        c1 = jnp.sum(is_cand.astype(jnp.int32), axis=(1, 2), keepdims=True)
        take = gcnt + c1 >= k
        prefix = jnp.where(take, cand, prefix)
        gcnt = jnp.where(take, gcnt, gcnt + c1)
        return prefix, gcnt

    z11 = jnp.zeros((B, 1, 1), jnp.int32)
    prefix, _ = jax.lax.fori_loop(0, 32, radix_body, (z11, z11))
    t_o = prefix ^ int_min

    gt = o > t_o
    eq = o == t_o
    G = jnp.sum(gt.astype(jnp.int32), axis=(1, 2), keepdims=True)
    needf = (k - G).astype(jnp.float32)
    m = jnp.max(L3, axis=(1, 2), keepdims=True)

    iota_c = jax.lax.broadcasted_iota(jnp.int32, (C, C), 1)
    iota_r = jax.lax.broadcasted_iota(jnp.int32, (C, C), 0)
    u_cols = (iota_r <= iota_c).astype(jnp.float32)
    ri = jax.lax.broadcasted_iota(jnp.int32, (R, R), 0)
    ci = jax.lax.broadcasted_iota(jnp.int32, (R, R), 1)
    l_rows = (ci < ri).astype(jnp.float32)
    idx = (jax.lax.broadcasted_iota(jnp.int32, (R, C), 0) * C
           + jax.lax.broadcasted_iota(jnp.int32, (R, C), 1)).astype(jnp.float32)
    iota_j = jax.lax.broadcasted_iota(jnp.int32, (k, C), 0).astype(jnp.float32)

    for b in range(B):
        rank_eq = _cumsum_excl(eq[b].astype(jnp.float32), u_cols, l_rows)
        sel = gt[b] | (eq[b] & (rank_eq < needf[b, 0, 0]))
        self32 = sel.astype(jnp.float32)
        p = _cumsum_excl(self32, u_cols, l_rows)
        e = jnp.exp(L3[b] - m[b, 0, 0]) * self32
        selidx = self32 * idx
        acc = jnp.zeros((2, k), jnp.float32)
        for r in range(R):
            oh = (iota_j == p[r:r + 1]).astype(jnp.float32)
            a2 = jnp.concatenate([selidx[r:r + 1], e[r:r + 1]], 0)
            acc = acc + jax.lax.dot_general(
                a2, oh, (((1,), (1,)), ((), ())),
                preferred_element_type=jnp.float32)
        z = jnp.sum(e)
        toks = acc[0:1].astype(jnp.int32)
        tok_ref[b] = toks
        rw_ref[b] = acc[1:2] / z
        gtok_ref[b] = toks + b * S


def _route(logits3, k):
    B, S, _ = logits3.shape
    R = 8
    C = S // R
    logr = logits3.reshape(B, R, C)
    tokens3, rw3, gtok3 = pl.pallas_call(
        functools.partial(_route_kernel, k=k, S=S),
        out_shape=[
            jax.ShapeDtypeStruct((B, 1, k), jnp.int32),
            jax.ShapeDtypeStruct((B, 1, k), jnp.float32),
            jax.ShapeDtypeStruct((B, 1, k), jnp.int32),
        ],
    )(logr)
    return tokens3.reshape(B, k), rw3.reshape(B, k, 1), gtok3.reshape(B * k)


def _sc_gather(xf, gtok, D):
    total = gtok.shape[0]
    NC, NS = 2, 16
    NW = NC * NS
    rows_w = total // NW
    CH = min(32, rows_w)
    nch = rows_w // CH
    mesh = plsc.VectorSubcoreMesh(core_axis_name="c", subcore_axis_name="s",
                                  num_cores=NC, num_subcores=NS)

    @functools.partial(
        pl.kernel,
        out_type=jax.ShapeDtypeStruct((total, D), jnp.float32),
        mesh=mesh,
        scratch_types=[
            pltpu.VMEM((CH,), jnp.int32),
            pltpu.VMEM((CH, D), jnp.float32),
            pltpu.SemaphoreType.DMA,
        ],
    )
    def gk(x_hbm, tok_hbm, out_hbm, idx_v, rows_v, sem):
        wid = jax.lax.axis_index("s") * NC + jax.lax.axis_index("c")
        base = wid * rows_w
        for c in range(nch):
            g0 = base + c * CH
            pltpu.sync_copy(tok_hbm.at[pl.ds(g0, CH)], idx_v)
            pltpu.async_copy(x_hbm.at[idx_v], rows_v, sem).wait()
            pltpu.sync_copy(rows_v, out_hbm.at[pl.ds(g0, CH), :])

    return gk(xf, gtok)


def _moe_kernel(tok_ref, filt_ref, wl_ref, bl_ref, rw_ref, res_any, out_any,
                ys, sems, *, bm, nsteps):
    b = pl.program_id(0)
    jb = pl.program_id(1)
    nj = pl.num_programs(1)
    s = b * nj + jb
    base = jb * bm
    buf = jax.lax.rem(s, 2)

    def _drain(which):
        def s_wait(j, _):
            pltpu.make_async_copy(ys.at[which, pl.ds(0, 1), :],
                                  out_any.at[0, pl.ds(0, 1), :],
                                  sems.at[which]).wait()
            return 0
        jax.lax.fori_loop(0, bm, s_wait, 0, unroll=16)

    @pl.when(s >= 2)
    def _():
        _drain(buf)

    xb = filt_ref[0]
    acc = jnp.dot(xb.astype(jnp.bfloat16),
                  wl_ref[...].astype(jnp.bfloat16),
                  preferred_element_type=jnp.float32)
    ys[buf] = xb + rw_ref[0] * (acc + bl_ref[...])

    def s_start(j, _):
        t = tok_ref[b, base + j]
        pltpu.make_async_copy(ys.at[buf, pl.ds(j, 1), :],
                              out_any.at[b, pl.ds(t, 1), :],
                              sems.at[buf]).start()
        return 0

    jax.lax.fori_loop(0, bm, s_start, 0, unroll=16)

    @pl.when(s == nsteps - 1)
    def _():
        _drain(buf)
        if nsteps >= 2:
            _drain(1 - buf)


def _moe(tokens, filt, x, Wl, bl2, rwk, res0, bm):
    B, S, D = x.shape
    k = tokens.shape[1]
    nsteps = B * (k // bm)
    grid_spec = pltpu.PrefetchScalarGridSpec(
        num_scalar_prefetch=1,
        grid=(B, k // bm),
        in_specs=[
            pl.BlockSpec((1, bm, D), lambda b, j, tok: (b, j, 0)),
            pl.BlockSpec((D, D), lambda b, j, tok: (0, 0)),
            pl.BlockSpec((1, D), lambda b, j, tok: (0, 0)),
            pl.BlockSpec((1, bm, 1), lambda b, j, tok: (b, j, 0)),
            pl.BlockSpec(memory_space=pl.MemorySpace.ANY),
        ],
        out_specs=pl.BlockSpec(memory_space=pl.MemorySpace.ANY),
        scratch_shapes=[
            pltpu.VMEM((2, bm, D), jnp.float32),
            pltpu.SemaphoreType.DMA((2,)),
        ],
    )
    return pl.pallas_call(
        functools.partial(_moe_kernel, bm=bm, nsteps=nsteps),
        grid_spec=grid_spec,
        out_shape=jax.ShapeDtypeStruct((B, S, D), jnp.float32),
        input_output_aliases={5: 0},
        compiler_params=pltpu.CompilerParams(
            dimension_semantics=("arbitrary", "arbitrary"),
        ),
    )(tokens, filt, Wl, bl2, rwk, res0)


def kernel(x, Wr, br, Wl, bl):
    B, S, D = x.shape
    k = int(S * SKIP) or 1
    res0, logits3 = _copy_and_logits(x, Wr, bs=1024)
    tokens, rwk, gtok = _route(logits3, k)
    filt = _sc_gather(x.reshape(B * S, D), gtok, D)
    bl2 = bl.reshape(1, D)
    return _moe(tokens, filt.reshape(B, k, D), x, Wl, bl2, rwk, res0, bm=512)
